# baseline (speedup 1.0000x reference)
import functools

import numpy as np
import jax
import jax.numpy as jnp
from jax import lax
from jax.experimental import pallas as pl
from jax.experimental.pallas import tpu as pltpu
from jax.experimental.pallas import tpu_sc as plsc

T = 1000
EMB = 128
BITS = 8
TPAD = 1024
ATBL = 16
SUB = 128

_BETAS = np.linspace(1e-4, 0.02, T, dtype=np.float32)
_AC = np.cumprod((1.0 - _BETAS).astype(np.float32)).astype(np.float32)
_AC_PAD = np.concatenate([_AC, np.zeros(TPAD - T, np.float32)]).reshape(1, TPAD)

_LN1E4 = float(np.log(10000.0))


def _seg_kernel(bfr_ref, pos_ref, eps_ref, tf_ref, ac_ref,
                aux_ref, condt_ref, alpha_ref, sigma_ref, bits_ref, sums_ref,
                *, blk, g, nblk):
    i = pl.program_id(0)

    @pl.when(i == 0)
    def _():
        sums_ref[...] = jnp.zeros_like(sums_ref)

    bfr = bfr_ref[0]
    gids = lax.broadcasted_iota(jnp.int32, (g, 1), 0).astype(jnp.float32)
    onehot_t = (gids == bfr).astype(jnp.float32).astype(jnp.bfloat16)
    ones = jnp.ones((blk, 1), jnp.float32)
    zeros = jnp.zeros((blk, 1), jnp.float32)
    vals = jnp.concatenate([pos_ref[...], eps_ref[...], ones, zeros],
                           axis=1).astype(jnp.bfloat16)
    sums_ref[...] += lax.dot_general(
        onehot_t, vals, (((1,), (0,)), ((), ())),
        preferred_element_type=jnp.float32)

    @pl.when(i == nblk - 1)
    def _():
        sums = sums_ref[...]
        counts = sums[:, 6:7]
        denom = jnp.maximum(counts, 1.0)
        mean = sums[:, 0:6] / denom

        tf = tf_ref[...]
        tiota = lax.broadcasted_iota(jnp.int32, (1, TPAD), 1).astype(jnp.float32)
        oh_t = (tf == tiota).astype(jnp.float32)
        ac_t = lax.dot_general(
            oh_t, ac_ref[...], (((1,), (1,)), ((), ())),
            preferred_element_type=jnp.float32)
        alpha = jnp.sqrt(ac_t)
        sigma = jnp.sqrt(1.0 - ac_t)
        alpha_ref[...] = alpha
        sigma_ref[...] = sigma

        half = EMB // 2
        fio = lax.broadcasted_iota(jnp.int32, (1, half), 1).astype(jnp.float32)
        freqs = jnp.exp(fio * (-_LN1E4 / half))
        args = tf * freqs
        condt_ref[...] = jnp.concatenate([jnp.sin(args), jnp.cos(args)], axis=1)

        ci = counts.astype(jnp.int32)
        bio = lax.broadcasted_iota(jnp.int32, (1, BITS), 1)
        bits_ref[...] = ((ci >> bio) & 1).astype(jnp.float32)

        pad = jnp.zeros((g, ATBL - 8), jnp.float32)
        aux_ref[...] = jnp.concatenate(
            [mean, alpha, sigma, pad], axis=1).astype(jnp.bfloat16)


def _atom_kernel(bf_ref, pos_ref, eps_ref, aux_ref, noisy_ref, eps_out_ref,
                 *, g):
    bf = bf_ref[...]
    gids = lax.broadcasted_iota(jnp.int32, (1, g), 1).astype(jnp.float32)
    onehot = (bf == gids).astype(jnp.float32).astype(jnp.bfloat16)
    gath = jnp.dot(onehot, aux_ref[...],
                   preferred_element_type=jnp.float32)
    mp = gath[:, 0:3]
    me = gath[:, 3:6]
    al = gath[:, 6:7]
    sg = gath[:, 7:8]
    x = pos_ref[...] - mp
    e = eps_ref[...] - me
    eps_out_ref[...] = e
    noisy_ref[...] = al * x + sg * e


def _make_sc_gather(n, g):
    info = plsc.get_sparse_core_info()
    nc, ns = info.num_cores, info.num_subcores
    nw = nc * ns
    nch = n // SUB
    tail = n - nch * SUB
    nch_all = nch + (1 if tail else 0)
    kmax = -(-nch_all // nw)
    mesh = plsc.VectorSubcoreMesh(core_axis_name="c", subcore_axis_name="s")

    scratch = [
        pltpu.VMEM((SUB,), jnp.int32),
        pltpu.VMEM((SUB, EMB), jnp.float32),
        pltpu.SemaphoreType.DMA,
    ]
    if tail:
        scratch += [
            pltpu.VMEM((tail,), jnp.int32),
            pltpu.VMEM((tail, EMB), jnp.float32),
        ]

    @functools.partial(
        pl.kernel, mesh=mesh,
        out_type=jax.ShapeDtypeStruct((n, EMB), jnp.float32),
        scratch_types=scratch,
    )
    def sc_gather(idx_hbm, table_hbm, out_hbm, idx_v, rows_v, sem,
                  *tail_scratch):
        wid = lax.axis_index("s") * nc + lax.axis_index("c")

        def body(k, carry):
            c = wid + nw * k

            @pl.when(c < nch)
            def _():
                base = c * SUB
                pltpu.sync_copy(idx_hbm.at[pl.ds(base, SUB)], idx_v)
                pltpu.async_copy(table_hbm.at[idx_v], rows_v, sem).wait()
                pltpu.sync_copy(rows_v, out_hbm.at[pl.ds(base, SUB)])

            if tail:
                idx_t, rows_t = tail_scratch

                @pl.when(c == nch)
                def _():
                    base = nch * SUB
                    pltpu.sync_copy(idx_hbm.at[pl.ds(base, tail)], idx_t)
                    pltpu.async_copy(table_hbm.at[idx_t], rows_t, sem).wait()
                    pltpu.sync_copy(rows_t, out_hbm.at[pl.ds(base, tail)])

            return carry

        lax.fori_loop(0, kmax, body, 0)

    return sc_gather


def _pick_blk(n, pref):
    for b in pref:
        if n % b == 0 and b % 8 == 0:
            return b
    return n


def kernel(pos, batch, eps_raw, t):
    n = pos.shape[0]
    g = t.shape[0]
    blk_s = _pick_blk(n, (2000, 1024, 1000, 512, 500, 256, 200, 128, 104, 8))
    nblk_s = n // blk_s
    blk_a = _pick_blk(n, (4000, 2000, 1024, 1000, 512, 500, 256, 200, 128, 104, 8))
    nblk_a = n // blk_a

    bf = batch.astype(jnp.float32).reshape(n, 1)
    bfr = batch.astype(jnp.float32).reshape(nblk_s, 1, blk_s)
    bi = batch.astype(jnp.int32)
    tf = t.astype(jnp.float32)
    ac = jnp.asarray(_AC_PAD)

    aux, condt, alpha, sigma, bits = pl.pallas_call(
        functools.partial(_seg_kernel, blk=blk_s, g=g, nblk=nblk_s),
        grid=(nblk_s,),
        in_specs=[
            pl.BlockSpec((1, 1, blk_s), lambda i: (i, 0, 0)),
            pl.BlockSpec((blk_s, 3), lambda i: (i, 0)),
            pl.BlockSpec((blk_s, 3), lambda i: (i, 0)),
            pl.BlockSpec((g, 1), lambda i: (0, 0)),
            pl.BlockSpec((1, TPAD), lambda i: (0, 0)),
        ],
        out_specs=[
            pl.BlockSpec((g, ATBL), lambda i: (0, 0)),
            pl.BlockSpec((g, EMB), lambda i: (0, 0)),
            pl.BlockSpec((g, 1), lambda i: (0, 0)),
            pl.BlockSpec((g, 1), lambda i: (0, 0)),
            pl.BlockSpec((g, BITS), lambda i: (0, 0)),
        ],
        out_shape=[
            jax.ShapeDtypeStruct((g, ATBL), jnp.bfloat16),
            jax.ShapeDtypeStruct((g, EMB), jnp.float32),
            jax.ShapeDtypeStruct((g, 1), jnp.float32),
            jax.ShapeDtypeStruct((g, 1), jnp.float32),
            jax.ShapeDtypeStruct((g, BITS), jnp.float32),
        ],
        scratch_shapes=[pltpu.VMEM((g, 8), jnp.float32)],
    )(bfr, pos, eps_raw, tf, ac)

    cond = _make_sc_gather(n, g)(bi, condt)

    noisy, eps = pl.pallas_call(
        functools.partial(_atom_kernel, g=g),
        grid=(nblk_a,),
        in_specs=[
            pl.BlockSpec((blk_a, 1), lambda i: (i, 0)),
            pl.BlockSpec((blk_a, 3), lambda i: (i, 0)),
            pl.BlockSpec((blk_a, 3), lambda i: (i, 0)),
            pl.BlockSpec((g, ATBL), lambda i: (0, 0)),
        ],
        out_specs=[
            pl.BlockSpec((blk_a, 3), lambda i: (i, 0)),
            pl.BlockSpec((blk_a, 3), lambda i: (i, 0)),
        ],
        out_shape=[
            jax.ShapeDtypeStruct((n, 3), jnp.float32),
            jax.ShapeDtypeStruct((n, 3), jnp.float32),
        ],
    )(bf, pos, eps_raw, aux)

    return (noisy, eps, cond, alpha, sigma, bits)


# speedup vs baseline: 1.0146x; 1.0146x over previous
import functools

import numpy as np
import jax
import jax.numpy as jnp
from jax import lax
from jax.experimental import pallas as pl
from jax.experimental.pallas import tpu as pltpu
from jax.experimental.pallas import tpu_sc as plsc

T = 1000
EMB = 128
BITS = 8
TPAD = 1024
ATBL = 16
SUB = 128

_BETAS = np.linspace(1e-4, 0.02, T, dtype=np.float32)
_AC = np.cumprod((1.0 - _BETAS).astype(np.float32)).astype(np.float32)
_AC_PAD = np.concatenate([_AC, np.zeros(TPAD - T, np.float32)]).reshape(1, TPAD)

_LN1E4 = float(np.log(10000.0))


def _seg_kernel(bfr_ref, pos_ref, eps_ref, tf_ref, ac_ref,
                aux_ref, condt_ref, alpha_ref, sigma_ref, bits_ref, sums_ref,
                *, blk, g, nblk):
    i = pl.program_id(0)

    @pl.when(i == 0)
    def _():
        sums_ref[...] = jnp.zeros_like(sums_ref)

    bfr = bfr_ref[0]
    gids = lax.broadcasted_iota(jnp.int32, (g, 1), 0).astype(jnp.float32)
    onehot_t = (gids == bfr).astype(jnp.float32).astype(jnp.bfloat16)
    ones = jnp.ones((blk, 1), jnp.float32)
    zeros = jnp.zeros((blk, 1), jnp.float32)
    vals = jnp.concatenate([pos_ref[...], eps_ref[...], ones, zeros],
                           axis=1).astype(jnp.bfloat16)
    sums_ref[...] += lax.dot_general(
        onehot_t, vals, (((1,), (0,)), ((), ())),
        preferred_element_type=jnp.float32)

    @pl.when(i == nblk - 1)
    def _():
        sums = sums_ref[...]
        counts = sums[:, 6:7]
        denom = jnp.maximum(counts, 1.0)
        mean = sums[:, 0:6] / denom

        tf = tf_ref[...]
        tiota = lax.broadcasted_iota(jnp.int32, (1, TPAD), 1).astype(jnp.float32)
        oh_t = (tf == tiota).astype(jnp.float32)
        ac_t = lax.dot_general(
            oh_t, ac_ref[...], (((1,), (1,)), ((), ())),
            preferred_element_type=jnp.float32)
        alpha = jnp.sqrt(ac_t)
        sigma = jnp.sqrt(1.0 - ac_t)
        alpha_ref[...] = alpha
        sigma_ref[...] = sigma

        half = EMB // 2
        fio = lax.broadcasted_iota(jnp.int32, (1, half), 1).astype(jnp.float32)
        freqs = jnp.exp(fio * (-_LN1E4 / half))
        args = tf * freqs
        condt_ref[...] = jnp.concatenate([jnp.sin(args), jnp.cos(args)], axis=1)

        ci = counts.astype(jnp.int32)
        bio = lax.broadcasted_iota(jnp.int32, (1, BITS), 1)
        bits_ref[...] = ((ci >> bio) & 1).astype(jnp.float32)

        pad = jnp.zeros((g, ATBL - 8), jnp.float32)
        aux_ref[...] = jnp.concatenate(
            [mean, alpha, sigma, pad], axis=1).astype(jnp.bfloat16)


def _atom_kernel(bf_ref, pos_ref, eps_ref, aux_ref, noisy_ref, eps_out_ref,
                 *, g):
    bf = bf_ref[...]
    gids = lax.broadcasted_iota(jnp.int32, (1, g), 1).astype(jnp.float32)
    onehot = (bf == gids).astype(jnp.float32).astype(jnp.bfloat16)
    gath = jnp.dot(onehot, aux_ref[...],
                   preferred_element_type=jnp.float32)
    mp = gath[:, 0:3]
    me = gath[:, 3:6]
    al = gath[:, 6:7]
    sg = gath[:, 7:8]
    x = pos_ref[...] - mp
    e = eps_ref[...] - me
    eps_out_ref[...] = e
    noisy_ref[...] = al * x + sg * e


def _make_sc_gather(n, g):
    info = plsc.get_sparse_core_info()
    nc, ns = info.num_cores, info.num_subcores
    nw = nc * ns
    nch = n // SUB
    tail = n - nch * SUB
    nch_all = nch + (1 if tail else 0)
    kpw = -(-nch_all // nw)
    wlen = kpw * SUB
    last_len = n - (nw - 1) * wlen
    w_t = nch // kpw
    j_t = nch % kpw
    mesh = plsc.VectorSubcoreMesh(core_axis_name="c", subcore_axis_name="s")

    scratch = [
        pltpu.VMEM((wlen,), jnp.int32),
        pltpu.VMEM((SUB, EMB), jnp.float32),
        pltpu.VMEM((SUB, EMB), jnp.float32),
        pltpu.SemaphoreType.DMA,
        pltpu.SemaphoreType.DMA,
        pltpu.SemaphoreType.DMA,
        pltpu.SemaphoreType.DMA,
    ]
    if tail:
        scratch += [
            pltpu.VMEM((tail, EMB), jnp.float32),
            pltpu.SemaphoreType.DMA,
        ]

    @functools.partial(
        pl.kernel, mesh=mesh,
        out_type=jax.ShapeDtypeStruct((n, EMB), jnp.float32),
        scratch_types=scratch,
    )
    def sc_gather(idx_hbm, table_hbm, out_hbm, idx_v, rows0, rows1,
                  gsem0, gsem1, wsem0, wsem1, *tail_scratch):
        wid = lax.axis_index("s") * nc + lax.axis_index("c")
        base_row = wid * wlen
        rows = (rows0, rows1)
        gsem = (gsem0, gsem1)
        wsem = (wsem0, wsem1)
        cntf = jnp.clip(nch - wid * kpw, 0, kpw)

        @pl.when(wid < nw - 1)
        def _():
            pltpu.sync_copy(idx_hbm.at[pl.ds(base_row, wlen)], idx_v)

        @pl.when(wid == nw - 1)
        def _():
            pltpu.sync_copy(idx_hbm.at[pl.ds(base_row, last_len)],
                            idx_v.at[pl.ds(0, last_len)])

        def body(jj, carry):
            for b in (0, 1):
                j = 2 * jj + b

                @pl.when(j < cntf)
                def _():
                    @pl.when(j >= 2)
                    def _():
                        pltpu.make_async_copy(
                            rows[b],
                            out_hbm.at[pl.ds(base_row + (j - 2) * SUB, SUB)],
                            wsem[b]).wait()

                    idx_sl = idx_v.at[pl.ds(j * SUB, SUB)]
                    pltpu.async_copy(
                        table_hbm.at[idx_sl], rows[b], gsem[b]).wait()
                    pltpu.async_copy(
                        rows[b],
                        out_hbm.at[pl.ds(base_row + j * SUB, SUB)],
                        wsem[b])

            return carry

        lax.fori_loop(0, (kpw + 1) // 2, body, 0)

        for b in (0, 1):
            @pl.when(cntf > b)
            def _():
                jl = ((cntf - 1 - b) // 2) * 2 + b
                pltpu.make_async_copy(
                    rows[b],
                    out_hbm.at[pl.ds(base_row + jl * SUB, SUB)],
                    wsem[b]).wait()

        if tail:
            rows_t, tsem = tail_scratch

            @pl.when(wid == w_t)
            def _():
                idx_sl = idx_v.at[pl.ds(j_t * SUB, tail)]
                pltpu.async_copy(table_hbm.at[idx_sl], rows_t, tsem).wait()
                pltpu.async_copy(
                    rows_t, out_hbm.at[pl.ds(nch * SUB, tail)], tsem).wait()

    return sc_gather


def _pick_blk(n, pref):
    for b in pref:
        if n % b == 0 and b % 8 == 0:
            return b
    return n


def kernel(pos, batch, eps_raw, t):
    n = pos.shape[0]
    g = t.shape[0]
    blk_s = _pick_blk(n, (2000, 1024, 1000, 512, 500, 256, 200, 128, 104, 8))
    nblk_s = n // blk_s
    blk_a = _pick_blk(n, (4000, 2000, 1024, 1000, 512, 500, 256, 200, 128, 104, 8))
    nblk_a = n // blk_a

    bf = batch.astype(jnp.float32).reshape(n, 1)
    bfr = batch.astype(jnp.float32).reshape(nblk_s, 1, blk_s)
    bi = batch.astype(jnp.int32)
    tf = t.astype(jnp.float32)
    ac = jnp.asarray(_AC_PAD)

    aux, condt, alpha, sigma, bits = pl.pallas_call(
        functools.partial(_seg_kernel, blk=blk_s, g=g, nblk=nblk_s),
        grid=(nblk_s,),
        in_specs=[
            pl.BlockSpec((1, 1, blk_s), lambda i: (i, 0, 0)),
            pl.BlockSpec((blk_s, 3), lambda i: (i, 0)),
            pl.BlockSpec((blk_s, 3), lambda i: (i, 0)),
            pl.BlockSpec((g, 1), lambda i: (0, 0)),
            pl.BlockSpec((1, TPAD), lambda i: (0, 0)),
        ],
        out_specs=[
            pl.BlockSpec((g, ATBL), lambda i: (0, 0)),
            pl.BlockSpec((g, EMB), lambda i: (0, 0)),
            pl.BlockSpec((g, 1), lambda i: (0, 0)),
            pl.BlockSpec((g, 1), lambda i: (0, 0)),
            pl.BlockSpec((g, BITS), lambda i: (0, 0)),
        ],
        out_shape=[
            jax.ShapeDtypeStruct((g, ATBL), jnp.bfloat16),
            jax.ShapeDtypeStruct((g, EMB), jnp.float32),
            jax.ShapeDtypeStruct((g, 1), jnp.float32),
            jax.ShapeDtypeStruct((g, 1), jnp.float32),
            jax.ShapeDtypeStruct((g, BITS), jnp.float32),
        ],
        scratch_shapes=[pltpu.VMEM((g, 8), jnp.float32)],
    )(bfr, pos, eps_raw, tf, ac)

    cond = _make_sc_gather(n, g)(bi, condt)

    noisy, eps = pl.pallas_call(
        functools.partial(_atom_kernel, g=g),
        grid=(nblk_a,),
        in_specs=[
            pl.BlockSpec((blk_a, 1), lambda i: (i, 0)),
            pl.BlockSpec((blk_a, 3), lambda i: (i, 0)),
            pl.BlockSpec((blk_a, 3), lambda i: (i, 0)),
            pl.BlockSpec((g, ATBL), lambda i: (0, 0)),
        ],
        out_specs=[
            pl.BlockSpec((blk_a, 3), lambda i: (i, 0)),
            pl.BlockSpec((blk_a, 3), lambda i: (i, 0)),
        ],
        out_shape=[
            jax.ShapeDtypeStruct((n, 3), jnp.float32),
            jax.ShapeDtypeStruct((n, 3), jnp.float32),
        ],
    )(bf, pos, eps_raw, aux)

    return (noisy, eps, cond, alpha, sigma, bits)


# speedup vs baseline: 1.4188x; 1.3983x over previous
import functools

import numpy as np
import jax
import jax.numpy as jnp
from jax import lax
from jax.experimental import pallas as pl
from jax.experimental.pallas import tpu as pltpu
from jax.experimental.pallas import tpu_sc as plsc

T = 1000
EMB = 128
BITS = 8
TPAD = 1024
ATBL = 16
SUB = 128

_BETAS = np.linspace(1e-4, 0.02, T, dtype=np.float32)
_AC = np.cumprod((1.0 - _BETAS).astype(np.float32)).astype(np.float32)
_AC_PAD = np.concatenate([_AC, np.zeros(TPAD - T, np.float32)]).reshape(1, TPAD)

_LN1E4 = float(np.log(10000.0))


def _condt_kernel(tf_ref, condt_ref, *, g):
    tf = tf_ref[...]
    half = EMB // 2
    fio = lax.broadcasted_iota(jnp.int32, (1, half), 1).astype(jnp.float32)
    freqs = jnp.exp(fio * (-_LN1E4 / half))
    args = tf * freqs
    condt_ref[...] = jnp.concatenate([jnp.sin(args), jnp.cos(args)], axis=1)


def _seg_kernel(bfr_ref, pos_ref, eps_ref, tf_ref, ac_ref,
                aux_ref, alpha_ref, sigma_ref, bits_ref, sums_ref,
                *, blk, g, nblk):
    i = pl.program_id(0)

    @pl.when(i == 0)
    def _():
        sums_ref[...] = jnp.zeros_like(sums_ref)

    bfr = bfr_ref[0]
    gids = lax.broadcasted_iota(jnp.int32, (g, 1), 0).astype(jnp.float32)
    onehot_t = (gids == bfr).astype(jnp.float32).astype(jnp.bfloat16)
    ones = jnp.ones((blk, 1), jnp.float32)
    zeros = jnp.zeros((blk, 1), jnp.float32)
    vals = jnp.concatenate([pos_ref[...], eps_ref[...], ones, zeros],
                           axis=1).astype(jnp.bfloat16)
    sums_ref[...] += lax.dot_general(
        onehot_t, vals, (((1,), (0,)), ((), ())),
        preferred_element_type=jnp.float32)

    @pl.when(i == nblk - 1)
    def _():
        sums = sums_ref[...]
        counts = sums[:, 6:7]
        denom = jnp.maximum(counts, 1.0)
        mean = sums[:, 0:6] / denom

        tf = tf_ref[...]
        tiota = lax.broadcasted_iota(jnp.int32, (1, TPAD), 1).astype(jnp.float32)
        oh_t = (tf == tiota).astype(jnp.float32)
        ac_t = lax.dot_general(
            oh_t, ac_ref[...], (((1,), (1,)), ((), ())),
            preferred_element_type=jnp.float32)
        alpha = jnp.sqrt(ac_t)
        sigma = jnp.sqrt(1.0 - ac_t)
        alpha_ref[...] = alpha
        sigma_ref[...] = sigma

        ci = counts.astype(jnp.int32)
        bio = lax.broadcasted_iota(jnp.int32, (1, BITS), 1)
        bits_ref[...] = ((ci >> bio) & 1).astype(jnp.float32)

        pad = jnp.zeros((g, ATBL - 8), jnp.float32)
        aux_ref[...] = jnp.concatenate(
            [mean, alpha, sigma, pad], axis=1).astype(jnp.bfloat16)


def _atom_kernel(bf_ref, pos_ref, eps_ref, aux_ref, noisy_ref, eps_out_ref,
                 *, g):
    bf = bf_ref[...]
    gids = lax.broadcasted_iota(jnp.int32, (1, g), 1).astype(jnp.float32)
    onehot = (bf == gids).astype(jnp.float32).astype(jnp.bfloat16)
    gath = jnp.dot(onehot, aux_ref[...],
                   preferred_element_type=jnp.float32)
    mp = gath[:, 0:3]
    me = gath[:, 3:6]
    al = gath[:, 6:7]
    sg = gath[:, 7:8]
    x = pos_ref[...] - mp
    e = eps_ref[...] - me
    eps_out_ref[...] = e
    noisy_ref[...] = al * x + sg * e


def _make_sc_gather(n, g):
    info = plsc.get_sparse_core_info()
    nc, ns = info.num_cores, info.num_subcores
    nw = nc * ns
    nch = n // SUB
    tail = n - nch * SUB
    nch_all = nch + (1 if tail else 0)
    kpw = -(-nch_all // nw)
    wlen = kpw * SUB
    last_len = n - (nw - 1) * wlen
    w_t = nch // kpw
    j_t = nch % kpw
    mesh = plsc.VectorSubcoreMesh(core_axis_name="c", subcore_axis_name="s")

    scratch = [
        pltpu.VMEM_SHARED((g, EMB), jnp.float32),
        pltpu.VMEM((wlen,), jnp.int32),
        pltpu.VMEM((SUB, EMB), jnp.float32),
        pltpu.VMEM((SUB, EMB), jnp.float32),
        pltpu.SemaphoreType.DMA,
        pltpu.SemaphoreType.DMA,
        pltpu.SemaphoreType.DMA,
        pltpu.SemaphoreType.DMA,
    ]
    if tail:
        scratch += [
            pltpu.VMEM((tail, EMB), jnp.float32),
            pltpu.SemaphoreType.DMA,
        ]

    @functools.partial(
        pl.kernel, mesh=mesh,
        out_type=jax.ShapeDtypeStruct((n, EMB), jnp.float32),
        scratch_types=scratch,
    )
    def sc_gather(idx_hbm, table_hbm, out_hbm, shared_tbl, idx_v, rows0,
                  rows1, gsem0, gsem1, wsem0, wsem1, *tail_scratch):
        sid = lax.axis_index("s")
        wid = sid * nc + lax.axis_index("c")

        @pl.when(sid == 0)
        def _():
            pltpu.sync_copy(table_hbm, shared_tbl)

        plsc.subcore_barrier()
        base_row = wid * wlen
        rows = (rows0, rows1)
        gsem = (gsem0, gsem1)
        wsem = (wsem0, wsem1)
        cntf = jnp.clip(nch - wid * kpw, 0, kpw)

        @pl.when(wid < nw - 1)
        def _():
            pltpu.sync_copy(idx_hbm.at[pl.ds(base_row, wlen)], idx_v)

        @pl.when(wid == nw - 1)
        def _():
            pltpu.sync_copy(idx_hbm.at[pl.ds(base_row, last_len)],
                            idx_v.at[pl.ds(0, last_len)])

        def body(jj, carry):
            for b in (0, 1):
                j = 2 * jj + b

                @pl.when(j < cntf)
                def _():
                    @pl.when(j >= 2)
                    def _():
                        pltpu.make_async_copy(
                            rows[b],
                            out_hbm.at[pl.ds(base_row + (j - 2) * SUB, SUB)],
                            wsem[b]).wait()

                    idx_sl = idx_v.at[pl.ds(j * SUB, SUB)]
                    pltpu.async_copy(
                        shared_tbl.at[idx_sl], rows[b], gsem[b]).wait()
                    pltpu.async_copy(
                        rows[b],
                        out_hbm.at[pl.ds(base_row + j * SUB, SUB)],
                        wsem[b])

            return carry

        lax.fori_loop(0, (kpw + 1) // 2, body, 0)

        for b in (0, 1):
            @pl.when(cntf > b)
            def _():
                jl = ((cntf - 1 - b) // 2) * 2 + b
                pltpu.make_async_copy(
                    rows[b],
                    out_hbm.at[pl.ds(base_row + jl * SUB, SUB)],
                    wsem[b]).wait()

        if tail:
            rows_t, tsem = tail_scratch

            @pl.when(wid == w_t)
            def _():
                idx_sl = idx_v.at[pl.ds(j_t * SUB, tail)]
                pltpu.async_copy(shared_tbl.at[idx_sl], rows_t, tsem).wait()
                pltpu.async_copy(
                    rows_t, out_hbm.at[pl.ds(nch * SUB, tail)], tsem).wait()

    return sc_gather


def _pick_blk(n, pref):
    for b in pref:
        if n % b == 0 and b % 8 == 0:
            return b
    return n


def kernel(pos, batch, eps_raw, t):
    n = pos.shape[0]
    g = t.shape[0]
    blk_s = _pick_blk(n, (2000, 1024, 1000, 512, 500, 256, 200, 128, 104, 8))
    nblk_s = n // blk_s
    blk_a = _pick_blk(n, (4000, 2000, 1024, 1000, 512, 500, 256, 200, 128, 104, 8))
    nblk_a = n // blk_a

    bf = batch.astype(jnp.float32).reshape(n, 1)
    bfr = batch.astype(jnp.float32).reshape(nblk_s, 1, blk_s)
    bi = batch.astype(jnp.int32)
    tf = t.astype(jnp.float32)
    ac = jnp.asarray(_AC_PAD)

    condt = pl.pallas_call(
        functools.partial(_condt_kernel, g=g),
        in_specs=[pl.BlockSpec((g, 1), lambda: (0, 0))],
        out_specs=pl.BlockSpec((g, EMB), lambda: (0, 0)),
        out_shape=jax.ShapeDtypeStruct((g, EMB), jnp.float32),
    )(tf)
    cond = _make_sc_gather(n, g)(bi, condt)

    aux, alpha, sigma, bits = pl.pallas_call(
        functools.partial(_seg_kernel, blk=blk_s, g=g, nblk=nblk_s),
        grid=(nblk_s,),
        in_specs=[
            pl.BlockSpec((1, 1, blk_s), lambda i: (i, 0, 0)),
            pl.BlockSpec((blk_s, 3), lambda i: (i, 0)),
            pl.BlockSpec((blk_s, 3), lambda i: (i, 0)),
            pl.BlockSpec((g, 1), lambda i: (0, 0)),
            pl.BlockSpec((1, TPAD), lambda i: (0, 0)),
        ],
        out_specs=[
            pl.BlockSpec((g, ATBL), lambda i: (0, 0)),
            pl.BlockSpec((g, 1), lambda i: (0, 0)),
            pl.BlockSpec((g, 1), lambda i: (0, 0)),
            pl.BlockSpec((g, BITS), lambda i: (0, 0)),
        ],
        out_shape=[
            jax.ShapeDtypeStruct((g, ATBL), jnp.bfloat16),
            jax.ShapeDtypeStruct((g, 1), jnp.float32),
            jax.ShapeDtypeStruct((g, 1), jnp.float32),
            jax.ShapeDtypeStruct((g, BITS), jnp.float32),
        ],
        scratch_shapes=[pltpu.VMEM((g, 8), jnp.float32)],
    )(bfr, pos, eps_raw, tf, ac)

    noisy, eps = pl.pallas_call(
        functools.partial(_atom_kernel, g=g),
        grid=(nblk_a,),
        in_specs=[
            pl.BlockSpec((blk_a, 1), lambda i: (i, 0)),
            pl.BlockSpec((blk_a, 3), lambda i: (i, 0)),
            pl.BlockSpec((blk_a, 3), lambda i: (i, 0)),
            pl.BlockSpec((g, ATBL), lambda i: (0, 0)),
        ],
        out_specs=[
            pl.BlockSpec((blk_a, 3), lambda i: (i, 0)),
            pl.BlockSpec((blk_a, 3), lambda i: (i, 0)),
        ],
        out_shape=[
            jax.ShapeDtypeStruct((n, 3), jnp.float32),
            jax.ShapeDtypeStruct((n, 3), jnp.float32),
        ],
    )(bf, pos, eps_raw, aux)

    return (noisy, eps, cond, alpha, sigma, bits)


# speedup vs baseline: 1.4680x; 1.0347x over previous
import functools

import numpy as np
import jax
import jax.numpy as jnp
from jax import lax
from jax.experimental import pallas as pl
from jax.experimental.pallas import tpu as pltpu
from jax.experimental.pallas import tpu_sc as plsc

T = 1000
EMB = 128
BITS = 8
TPAD = 1024
ATBL = 16
SUB = 128

_BETAS = np.linspace(1e-4, 0.02, T, dtype=np.float32)
_AC = np.cumprod((1.0 - _BETAS).astype(np.float32)).astype(np.float32)
_AC_PAD = np.concatenate([_AC, np.zeros(TPAD - T, np.float32)]).reshape(1, TPAD)

_LN1E4 = float(np.log(10000.0))


def _condt_kernel(tf_ref, condt_ref, *, g):
    tf = tf_ref[...]
    half = EMB // 2
    fio = lax.broadcasted_iota(jnp.int32, (1, half), 1).astype(jnp.float32)
    freqs = jnp.exp(fio * (-_LN1E4 / half))
    args = tf * freqs
    condt_ref[...] = jnp.concatenate([jnp.sin(args), jnp.cos(args)], axis=1)


def _seg_kernel(bfr_ref, pos_ref, eps_ref, tf_ref, ac_ref,
                aux_ref, alpha_ref, sigma_ref, bits_ref, sums_ref,
                *, blk, g, nblk):
    i = pl.program_id(0)

    @pl.when(i == 0)
    def _():
        sums_ref[...] = jnp.zeros_like(sums_ref)

    bfr = bfr_ref[0]
    gids = lax.broadcasted_iota(jnp.int32, (g, 1), 0).astype(jnp.float32)
    onehot_t = (gids == bfr).astype(jnp.float32).astype(jnp.bfloat16)
    ones = jnp.ones((blk, 1), jnp.float32)
    zeros = jnp.zeros((blk, 1), jnp.float32)
    vals = jnp.concatenate([pos_ref[...], eps_ref[...], ones, zeros],
                           axis=1).astype(jnp.bfloat16)
    sums_ref[...] += lax.dot_general(
        onehot_t, vals, (((1,), (0,)), ((), ())),
        preferred_element_type=jnp.float32)

    @pl.when(i == nblk - 1)
    def _():
        sums = sums_ref[...]
        counts = sums[:, 6:7]
        denom = jnp.maximum(counts, 1.0)
        mean = sums[:, 0:6] / denom

        tf = tf_ref[...]
        tiota = lax.broadcasted_iota(jnp.int32, (1, TPAD), 1).astype(jnp.float32)
        oh_t = (tf == tiota).astype(jnp.float32)
        ac_t = lax.dot_general(
            oh_t, ac_ref[...], (((1,), (1,)), ((), ())),
            preferred_element_type=jnp.float32)
        alpha = jnp.sqrt(ac_t)
        sigma = jnp.sqrt(1.0 - ac_t)
        alpha_ref[...] = alpha
        sigma_ref[...] = sigma

        ci = counts.astype(jnp.int32)
        bio = lax.broadcasted_iota(jnp.int32, (1, BITS), 1)
        bits_ref[...] = ((ci >> bio) & 1).astype(jnp.float32)

        pad = jnp.zeros((g, ATBL - 8), jnp.float32)
        aux_ref[...] = jnp.concatenate(
            [mean, alpha, sigma, pad], axis=1).astype(jnp.bfloat16)


def _atom_kernel(bf_ref, pos_ref, eps_ref, aux_ref, noisy_ref, eps_out_ref,
                 *, g):
    bf = bf_ref[...]
    gids = lax.broadcasted_iota(jnp.int32, (1, g), 1).astype(jnp.float32)
    onehot = (bf == gids).astype(jnp.float32).astype(jnp.bfloat16)
    gath = jnp.dot(onehot, aux_ref[...],
                   preferred_element_type=jnp.float32)
    mp = gath[:, 0:3]
    me = gath[:, 3:6]
    al = gath[:, 6:7]
    sg = gath[:, 7:8]
    x = pos_ref[...] - mp
    e = eps_ref[...] - me
    eps_out_ref[...] = e
    noisy_ref[...] = al * x + sg * e


def _make_sc_gather(n, g):
    info = plsc.get_sparse_core_info()
    nc, ns = info.num_cores, info.num_subcores
    nw = nc * ns
    nch = n // SUB
    tail = n - nch * SUB
    nch_all = nch + (1 if tail else 0)
    kpw = -(-nch_all // nw)
    wlen = kpw * SUB
    last_len = n - (nw - 1) * wlen
    w_t = nch // kpw
    j_t = nch % kpw
    mesh = plsc.VectorSubcoreMesh(core_axis_name="c", subcore_axis_name="s")

    scratch = [
        pltpu.VMEM_SHARED((g, EMB), jnp.float32),
        pltpu.VMEM((wlen,), jnp.int32),
        pltpu.VMEM((SUB, EMB), jnp.float32),
        pltpu.VMEM((SUB, EMB), jnp.float32),
        pltpu.SemaphoreType.DMA,
        pltpu.SemaphoreType.DMA,
        pltpu.SemaphoreType.DMA,
        pltpu.SemaphoreType.DMA,
    ]
    if tail:
        scratch += [
            pltpu.VMEM((tail, EMB), jnp.float32),
            pltpu.SemaphoreType.DMA,
        ]

    @functools.partial(
        pl.kernel, mesh=mesh,
        out_type=jax.ShapeDtypeStruct((n, EMB), jnp.float32),
        scratch_types=scratch,
    )
    def sc_gather(idx_hbm, table_hbm, out_hbm, shared_tbl, idx_v, rows0,
                  rows1, gsem0, gsem1, wsem0, wsem1, *tail_scratch):
        sid = lax.axis_index("s")
        wid = sid * nc + lax.axis_index("c")

        @pl.when(sid == 0)
        def _():
            pltpu.sync_copy(table_hbm, shared_tbl)

        plsc.subcore_barrier()
        base_row = wid * wlen
        rows = (rows0, rows1)
        gsem = (gsem0, gsem1)
        wsem = (wsem0, wsem1)
        cntf = jnp.clip(nch - wid * kpw, 0, kpw)

        @pl.when(wid < nw - 1)
        def _():
            pltpu.sync_copy(idx_hbm.at[pl.ds(base_row, wlen)], idx_v)

        @pl.when(wid == nw - 1)
        def _():
            pltpu.sync_copy(idx_hbm.at[pl.ds(base_row, last_len)],
                            idx_v.at[pl.ds(0, last_len)])

        def body(jj, carry):
            for b in (0, 1):
                j = 2 * jj + b

                @pl.when(j < cntf)
                def _():
                    @pl.when(j >= 2)
                    def _():
                        pltpu.make_async_copy(
                            rows[b],
                            out_hbm.at[pl.ds(base_row + (j - 2) * SUB, SUB)],
                            wsem[b]).wait()

                    idx_sl = idx_v.at[pl.ds(j * SUB, SUB)]
                    pltpu.async_copy(
                        shared_tbl.at[idx_sl], rows[b], gsem[b]).wait()
                    pltpu.async_copy(
                        rows[b],
                        out_hbm.at[pl.ds(base_row + j * SUB, SUB)],
                        wsem[b])

            return carry

        lax.fori_loop(0, (kpw + 1) // 2, body, 0)

        for b in (0, 1):
            @pl.when(cntf > b)
            def _():
                jl = ((cntf - 1 - b) // 2) * 2 + b
                pltpu.make_async_copy(
                    rows[b],
                    out_hbm.at[pl.ds(base_row + jl * SUB, SUB)],
                    wsem[b]).wait()

        if tail:
            rows_t, tsem = tail_scratch

            @pl.when(wid == w_t)
            def _():
                idx_sl = idx_v.at[pl.ds(j_t * SUB, tail)]
                pltpu.async_copy(shared_tbl.at[idx_sl], rows_t, tsem).wait()
                pltpu.async_copy(
                    rows_t, out_hbm.at[pl.ds(nch * SUB, tail)], tsem).wait()

    return sc_gather


def _pick_blk(n, pref):
    for b in pref:
        if n % b == 0 and b % 8 == 0:
            return b
    return n


def kernel(pos, batch, eps_raw, t):
    n = pos.shape[0]
    g = t.shape[0]
    blk_s = _pick_blk(n, (4000, 2000, 1024, 1000, 512, 500, 256, 200, 128, 104, 8))
    nblk_s = n // blk_s
    blk_a = _pick_blk(n, (4000, 2000, 1024, 1000, 512, 500, 256, 200, 128, 104, 8))
    nblk_a = n // blk_a

    bf = batch.astype(jnp.float32).reshape(n, 1)
    bfr = batch.astype(jnp.float32).reshape(nblk_s, 1, blk_s)
    bi = batch.astype(jnp.int32)
    tf = t.astype(jnp.float32)
    ac = jnp.asarray(_AC_PAD)

    condt = pl.pallas_call(
        functools.partial(_condt_kernel, g=g),
        in_specs=[pl.BlockSpec((g, 1), lambda: (0, 0))],
        out_specs=pl.BlockSpec((g, EMB), lambda: (0, 0)),
        out_shape=jax.ShapeDtypeStruct((g, EMB), jnp.float32),
    )(tf)
    cond = _make_sc_gather(n, g)(bi, condt)

    aux, alpha, sigma, bits = pl.pallas_call(
        functools.partial(_seg_kernel, blk=blk_s, g=g, nblk=nblk_s),
        grid=(nblk_s,),
        in_specs=[
            pl.BlockSpec((1, 1, blk_s), lambda i: (i, 0, 0)),
            pl.BlockSpec((blk_s, 3), lambda i: (i, 0)),
            pl.BlockSpec((blk_s, 3), lambda i: (i, 0)),
            pl.BlockSpec((g, 1), lambda i: (0, 0)),
            pl.BlockSpec((1, TPAD), lambda i: (0, 0)),
        ],
        out_specs=[
            pl.BlockSpec((g, ATBL), lambda i: (0, 0)),
            pl.BlockSpec((g, 1), lambda i: (0, 0)),
            pl.BlockSpec((g, 1), lambda i: (0, 0)),
            pl.BlockSpec((g, BITS), lambda i: (0, 0)),
        ],
        out_shape=[
            jax.ShapeDtypeStruct((g, ATBL), jnp.bfloat16),
            jax.ShapeDtypeStruct((g, 1), jnp.float32),
            jax.ShapeDtypeStruct((g, 1), jnp.float32),
            jax.ShapeDtypeStruct((g, BITS), jnp.float32),
        ],
        scratch_shapes=[pltpu.VMEM((g, 8), jnp.float32)],
    )(bfr, pos, eps_raw, tf, ac)

    noisy, eps = pl.pallas_call(
        functools.partial(_atom_kernel, g=g),
        grid=(nblk_a,),
        in_specs=[
            pl.BlockSpec((blk_a, 1), lambda i: (i, 0)),
            pl.BlockSpec((blk_a, 3), lambda i: (i, 0)),
            pl.BlockSpec((blk_a, 3), lambda i: (i, 0)),
            pl.BlockSpec((g, ATBL), lambda i: (0, 0)),
        ],
        out_specs=[
            pl.BlockSpec((blk_a, 3), lambda i: (i, 0)),
            pl.BlockSpec((blk_a, 3), lambda i: (i, 0)),
        ],
        out_shape=[
            jax.ShapeDtypeStruct((n, 3), jnp.float32),
            jax.ShapeDtypeStruct((n, 3), jnp.float32),
        ],
    )(bf, pos, eps_raw, aux)

    return (noisy, eps, cond, alpha, sigma, bits)


# speedup vs baseline: 1.4891x; 1.0144x over previous
import functools

import numpy as np
import jax
import jax.numpy as jnp
from jax import lax
from jax.experimental import pallas as pl
from jax.experimental.pallas import tpu as pltpu
from jax.experimental.pallas import tpu_sc as plsc

T = 1000
EMB = 128
BITS = 8
TPAD = 1024
ATBL = 16
SUB = 128

_BETAS = np.linspace(1e-4, 0.02, T, dtype=np.float32)
_AC = np.cumprod((1.0 - _BETAS).astype(np.float32)).astype(np.float32)
_AC_PAD = np.concatenate([_AC, np.zeros(TPAD - T, np.float32)]).reshape(1, TPAD)

_LN1E4 = float(np.log(10000.0))


def _condt_kernel(tf_ref, condt_ref, *, g):
    tf = tf_ref[...]
    half = EMB // 2
    fio = lax.broadcasted_iota(jnp.int32, (1, half), 1).astype(jnp.float32)
    freqs = jnp.exp(fio * (-_LN1E4 / half))
    args = tf * freqs
    condt_ref[...] = jnp.concatenate([jnp.sin(args), jnp.cos(args)], axis=1)


def _seg_kernel(bfr_ref, pos_ref, eps_ref, tf_ref, ac_ref,
                aux_ref, alpha_ref, sigma_ref, bits_ref, sums_ref,
                *, blk, g, nblk):
    i = pl.program_id(0)

    @pl.when(i == 0)
    def _():
        sums_ref[...] = jnp.zeros_like(sums_ref)

    bfr = bfr_ref[0]
    gids = lax.broadcasted_iota(jnp.int32, (g, 1), 0).astype(jnp.float32)
    onehot_t = (gids == bfr).astype(jnp.float32).astype(jnp.bfloat16)
    ones = jnp.ones((blk, 1), jnp.float32)
    zeros = jnp.zeros((blk, 1), jnp.float32)
    vals = jnp.concatenate([pos_ref[...], eps_ref[...], ones, zeros],
                           axis=1).astype(jnp.bfloat16)
    sums_ref[...] += lax.dot_general(
        onehot_t, vals, (((1,), (0,)), ((), ())),
        preferred_element_type=jnp.float32)

    @pl.when(i == nblk - 1)
    def _():
        sums = sums_ref[...]
        counts = sums[:, 6:7]
        denom = jnp.maximum(counts, 1.0)
        mean = sums[:, 0:6] / denom

        tf = tf_ref[...]
        tiota = lax.broadcasted_iota(jnp.int32, (1, TPAD), 1).astype(jnp.float32)
        oh_t = (tf == tiota).astype(jnp.float32)
        ac_t = lax.dot_general(
            oh_t, ac_ref[...], (((1,), (1,)), ((), ())),
            preferred_element_type=jnp.float32)
        alpha = jnp.sqrt(ac_t)
        sigma = jnp.sqrt(1.0 - ac_t)
        alpha_ref[...] = alpha
        sigma_ref[...] = sigma

        ci = counts.astype(jnp.int32)
        bio = lax.broadcasted_iota(jnp.int32, (1, BITS), 1)
        bits_ref[...] = ((ci >> bio) & 1).astype(jnp.float32)

        pad = jnp.zeros((g, ATBL - 8), jnp.float32)
        aux_ref[...] = jnp.concatenate(
            [mean, alpha, sigma, pad], axis=1).astype(jnp.bfloat16)


def _atom_kernel(bf_ref, pos_ref, eps_ref, aux_ref, noisy_ref, eps_out_ref,
                 *, g):
    bf = bf_ref[...]
    gids = lax.broadcasted_iota(jnp.int32, (1, g), 1).astype(jnp.float32)
    onehot = (bf == gids).astype(jnp.float32).astype(jnp.bfloat16)
    gath = jnp.dot(onehot, aux_ref[...],
                   preferred_element_type=jnp.float32)
    mp = gath[:, 0:3]
    me = gath[:, 3:6]
    al = gath[:, 6:7]
    sg = gath[:, 7:8]
    x = pos_ref[...] - mp
    e = eps_ref[...] - me
    eps_out_ref[...] = e
    noisy_ref[...] = al * x + sg * e


def _make_sc_gather(n, g):
    info = plsc.get_sparse_core_info()
    nc, ns = info.num_cores, info.num_subcores
    nw = nc * ns
    nch = n // SUB
    tail = n - nch * SUB
    nch_all = nch + (1 if tail else 0)
    kpw = -(-nch_all // nw)
    wlen = kpw * SUB
    last_len = n - (nw - 1) * wlen
    w_t = nch // kpw
    j_t = nch % kpw
    mesh = plsc.VectorSubcoreMesh(core_axis_name="c", subcore_axis_name="s")

    scratch = [
        pltpu.VMEM_SHARED((g, EMB), jnp.float32),
        pltpu.VMEM((wlen,), jnp.int32),
        pltpu.VMEM((SUB, EMB), jnp.float32),
        pltpu.VMEM((SUB, EMB), jnp.float32),
        pltpu.SemaphoreType.DMA,
        pltpu.SemaphoreType.DMA,
        pltpu.SemaphoreType.DMA,
        pltpu.SemaphoreType.DMA,
    ]
    if tail:
        scratch += [
            pltpu.VMEM((tail, EMB), jnp.float32),
            pltpu.SemaphoreType.DMA,
        ]

    @functools.partial(
        pl.kernel, mesh=mesh,
        out_type=jax.ShapeDtypeStruct((n, EMB), jnp.float32),
        scratch_types=scratch,
    )
    def sc_gather(idx_hbm, table_hbm, out_hbm, shared_tbl, idx_v, rows0,
                  rows1, gsem0, gsem1, wsem0, wsem1, *tail_scratch):
        sid = lax.axis_index("s")
        wid = sid * nc + lax.axis_index("c")

        @pl.when(sid == 0)
        def _():
            pltpu.sync_copy(table_hbm, shared_tbl)

        plsc.subcore_barrier()
        base_row = wid * wlen
        rows = (rows0, rows1)
        gsem = (gsem0, gsem1)
        wsem = (wsem0, wsem1)
        cntf = jnp.clip(nch - wid * kpw, 0, kpw)

        @pl.when(wid < nw - 1)
        def _():
            pltpu.sync_copy(idx_hbm.at[pl.ds(base_row, wlen)], idx_v)

        @pl.when(wid == nw - 1)
        def _():
            pltpu.sync_copy(idx_hbm.at[pl.ds(base_row, last_len)],
                            idx_v.at[pl.ds(0, last_len)])

        def body(jj, carry):
            for b in (0, 1):
                j = 2 * jj + b

                @pl.when(j < cntf)
                def _():
                    @pl.when(j >= 2)
                    def _():
                        pltpu.make_async_copy(
                            rows[b],
                            out_hbm.at[pl.ds(base_row + (j - 2) * SUB, SUB)],
                            wsem[b]).wait()

                    idx_sl = idx_v.at[pl.ds(j * SUB, SUB)]
                    pltpu.async_copy(
                        shared_tbl.at[idx_sl], rows[b], gsem[b]).wait()
                    pltpu.async_copy(
                        rows[b],
                        out_hbm.at[pl.ds(base_row + j * SUB, SUB)],
                        wsem[b])

            return carry

        lax.fori_loop(0, (kpw + 1) // 2, body, 0)

        for b in (0, 1):
            @pl.when(cntf > b)
            def _():
                jl = ((cntf - 1 - b) // 2) * 2 + b
                pltpu.make_async_copy(
                    rows[b],
                    out_hbm.at[pl.ds(base_row + jl * SUB, SUB)],
                    wsem[b]).wait()

        if tail:
            rows_t, tsem = tail_scratch

            @pl.when(wid == w_t)
            def _():
                idx_sl = idx_v.at[pl.ds(j_t * SUB, tail)]
                pltpu.async_copy(shared_tbl.at[idx_sl], rows_t, tsem).wait()
                pltpu.async_copy(
                    rows_t, out_hbm.at[pl.ds(nch * SUB, tail)], tsem).wait()

    return sc_gather


def _pick_blk(n, pref):
    for b in pref:
        if n % b == 0 and b % 8 == 0:
            return b
    return n


def kernel(pos, batch, eps_raw, t):
    n = pos.shape[0]
    g = t.shape[0]
    blk_s = _pick_blk(n, (5000, 4000, 2000, 1024, 1000, 512, 500, 256, 200, 128, 104, 8))
    nblk_s = n // blk_s
    blk_a = _pick_blk(n, (5000, 4000, 2000, 1024, 1000, 512, 500, 256, 200, 128, 104, 8))
    nblk_a = n // blk_a

    bf = batch.astype(jnp.float32).reshape(n, 1)
    bfr = batch.astype(jnp.float32).reshape(nblk_s, 1, blk_s)
    bi = batch.astype(jnp.int32)
    tf = t.astype(jnp.float32)
    ac = jnp.asarray(_AC_PAD)

    condt = pl.pallas_call(
        functools.partial(_condt_kernel, g=g),
        in_specs=[pl.BlockSpec((g, 1), lambda: (0, 0))],
        out_specs=pl.BlockSpec((g, EMB), lambda: (0, 0)),
        out_shape=jax.ShapeDtypeStruct((g, EMB), jnp.float32),
    )(tf)
    cond = _make_sc_gather(n, g)(bi, condt)

    aux, alpha, sigma, bits = pl.pallas_call(
        functools.partial(_seg_kernel, blk=blk_s, g=g, nblk=nblk_s),
        grid=(nblk_s,),
        in_specs=[
            pl.BlockSpec((1, 1, blk_s), lambda i: (i, 0, 0)),
            pl.BlockSpec((blk_s, 3), lambda i: (i, 0)),
            pl.BlockSpec((blk_s, 3), lambda i: (i, 0)),
            pl.BlockSpec((g, 1), lambda i: (0, 0)),
            pl.BlockSpec((1, TPAD), lambda i: (0, 0)),
        ],
        out_specs=[
            pl.BlockSpec((g, ATBL), lambda i: (0, 0)),
            pl.BlockSpec((g, 1), lambda i: (0, 0)),
            pl.BlockSpec((g, 1), lambda i: (0, 0)),
            pl.BlockSpec((g, BITS), lambda i: (0, 0)),
        ],
        out_shape=[
            jax.ShapeDtypeStruct((g, ATBL), jnp.bfloat16),
            jax.ShapeDtypeStruct((g, 1), jnp.float32),
            jax.ShapeDtypeStruct((g, 1), jnp.float32),
            jax.ShapeDtypeStruct((g, BITS), jnp.float32),
        ],
        scratch_shapes=[pltpu.VMEM((g, 8), jnp.float32)],
    )(bfr, pos, eps_raw, tf, ac)

    noisy, eps = pl.pallas_call(
        functools.partial(_atom_kernel, g=g),
        grid=(nblk_a,),
        in_specs=[
            pl.BlockSpec((blk_a, 1), lambda i: (i, 0)),
            pl.BlockSpec((blk_a, 3), lambda i: (i, 0)),
            pl.BlockSpec((blk_a, 3), lambda i: (i, 0)),
            pl.BlockSpec((g, ATBL), lambda i: (0, 0)),
        ],
        out_specs=[
            pl.BlockSpec((blk_a, 3), lambda i: (i, 0)),
            pl.BlockSpec((blk_a, 3), lambda i: (i, 0)),
        ],
        out_shape=[
            jax.ShapeDtypeStruct((n, 3), jnp.float32),
            jax.ShapeDtypeStruct((n, 3), jnp.float32),
        ],
    )(bf, pos, eps_raw, aux)

    return (noisy, eps, cond, alpha, sigma, bits)
